# baseline (speedup 1.0000x reference)
import functools

import jax
import jax.numpy as jnp
from jax import lax
from jax.experimental import pallas as pl
from jax.experimental.pallas import tpu as pltpu
from jax.experimental.pallas import tpu_sc as plsc

SIGMA = 1.0

_B = 1024
_N = 200
_NC = 2
_NS = 16
_NW = _NC * _NS
_RPW = _B // _NW
_FPW = _RPW * _N
_NCHUNK = 13


def _sc_body(sims_hbm, levs_hbm, out_hbm, s_v, l_v, d_v, o_v):
    wid = lax.axis_index("s") * _NC + lax.axis_index("c")
    base = wid * _FPW

    pltpu.sync_copy(sims_hbm.at[pl.ds(base, _FPW)], s_v.at[pl.ds(0, _FPW)])
    pltpu.sync_copy(levs_hbm.at[pl.ds(base, _FPW)], l_v.at[pl.ds(0, _FPW)])

    def d_chunk(i, carry):
        o = i * 16
        d_v[pl.ds(o, 16)] = s_v[pl.ds(o, 16)] - SIGMA * l_v[pl.ds(o, 16)]
        return carry
    lax.fori_loop(0, _FPW // 16, d_chunk, 0)

    iota = lax.iota(jnp.int32, 16)
    zero16 = jnp.zeros((16,), jnp.float32)
    tailmask = iota >= 8

    def p_body(rbase, p, accs):
        dp = jnp.full((16,), d_v[pl.ds(rbase + p, 16)][0], jnp.float32)
        lp = jnp.full((16,), l_v[pl.ds(rbase + p, 16)][0], jnp.float32)
        pb = jnp.full((16,), p, jnp.int32)
        new = []
        for c in range(_NCHUNK):
            off = 16 * c if c < 12 else _N - 16
            dq = d_v[pl.ds(rbase + off, 16)]
            lq = l_v[pl.ds(rbase + off, 16)]
            qi = iota + off
            t = jnp.maximum(dp - dq, 0.0)
            m = (lp < lq) | ((lp == lq) & (pb < qi))
            if c == 12:
                m = m & tailmask
            new.append(accs[c] + jnp.where(m, t, zero16))
        return tuple(new)

    def row_body(r, accs):
        return lax.fori_loop(0, _N, functools.partial(p_body, r * _N), accs)

    accs = lax.fori_loop(0, _RPW, row_body, (zero16,) * _NCHUNK)
    total = accs[0]
    for c in range(1, _NCHUNK):
        total = total + accs[c]
    o_v[...] = total
    pltpu.sync_copy(o_v, out_hbm.at[wid])


@jax.jit
def _sc_pairwise(similarities, levs):
    mesh = plsc.VectorSubcoreMesh(core_axis_name="c", subcore_axis_name="s")
    f = functools.partial(
        pl.kernel,
        out_type=jax.ShapeDtypeStruct((_NW, 16), jnp.float32),
        mesh=mesh,
        scratch_types=[
            pltpu.VMEM((_FPW + 16,), jnp.float32),
            pltpu.VMEM((_FPW + 16,), jnp.float32),
            pltpu.VMEM((_FPW + 16,), jnp.float32),
            pltpu.VMEM((16,), jnp.float32),
        ],
    )(_sc_body)
    return f(similarities.reshape(-1), levs.reshape(-1))


def kernel(similarities, levs):
    levs = levs.reshape(similarities.shape)
    partials = _sc_pairwise(similarities, levs)
    return jnp.sum(partials) / jnp.float32(_B * _N * _N)


# speedup vs baseline: 2.1737x; 2.1737x over previous
import functools

import jax
import jax.numpy as jnp
from jax import lax
from jax.experimental import pallas as pl
from jax.experimental.pallas import tpu as pltpu
from jax.experimental.pallas import tpu_sc as plsc

SIGMA = 1.0

_B = 1024
_N = 200
_NC = 2
_NS = 16
_NW = _NC * _NS
_RPW = _B // _NW
_FPW = _RPW * _N
_NCHUNK = 13
_NP = _NCHUNK * 16
_INF = float("inf")


def _sc_body(sims_hbm, levs_hbm, out_hbm, ss_v, sl_v, d_v, l_v, o_v):
    wid = lax.axis_index("s") * _NC + lax.axis_index("c")
    base = wid * _FPW

    pltpu.sync_copy(sims_hbm.at[pl.ds(base, _FPW)], ss_v.at[pl.ds(0, _FPW)])
    pltpu.sync_copy(levs_hbm.at[pl.ds(base, _FPW)], sl_v.at[pl.ds(0, _FPW)])

    iota = lax.iota(jnp.int32, 16)
    head8 = iota < 8

    def lay_row(r, carry):
        src = r * _N
        dst = r * _NP
        for c in range(12):
            sv = ss_v[pl.ds(src + 16 * c, 16)]
            lv = sl_v[pl.ds(src + 16 * c, 16)]
            d_v[pl.ds(dst + 16 * c, 16)] = sv - SIGMA * lv
            l_v[pl.ds(dst + 16 * c, 16)] = lv
        sv = ss_v[pl.ds(src + 192, 16)]
        lv = sl_v[pl.ds(src + 192, 16)]
        d_v[pl.ds(dst + 192, 16)] = jnp.where(head8, sv - SIGMA * lv, _INF)
        l_v[pl.ds(dst + 192, 16)] = jnp.where(head8, lv, _INF)
        return carry
    lax.fori_loop(0, _RPW, lay_row, 0)

    zero16 = jnp.zeros((16,), jnp.float32)

    def row_body(r, accs):
        rbase = r * _NP
        dqs = [d_v[pl.ds(rbase + 16 * c, 16)] for c in range(_NCHUNK)]
        lqs = [l_v[pl.ds(rbase + 16 * c, 16)] for c in range(_NCHUNK)]

        for cp in range(_NCHUNK):
            def i_body(i, accs, cp=cp):
                pa = rbase + 16 * cp + i
                dp = jnp.full((16,), d_v[pl.ds(pa, 16)][0], jnp.float32)
                lp = jnp.full((16,), l_v[pl.ds(pa, 16)][0], jnp.float32)
                new = list(accs)
                t = jnp.abs(dp - dqs[cp])
                m = jnp.logical_xor(lp <= lqs[cp], dp <= dqs[cp])
                m = m & (iota > jnp.full((16,), i, jnp.int32))
                new[cp] = new[cp] + jnp.where(m, t, zero16)
                for cq in range(cp + 1, _NCHUNK):
                    t = jnp.abs(dp - dqs[cq])
                    m = jnp.logical_xor(lp <= lqs[cq], dp <= dqs[cq])
                    new[cq] = new[cq] + jnp.where(m, t, zero16)
                return tuple(new)
            accs = lax.fori_loop(0, 16, i_body, accs)
        return accs

    accs = lax.fori_loop(0, _RPW, row_body, (zero16,) * _NCHUNK)
    total = accs[0]
    for c in range(1, _NCHUNK):
        total = total + accs[c]
    o_v[...] = total
    pltpu.sync_copy(o_v, out_hbm.at[wid])


@jax.jit
def _sc_pairwise(similarities, levs):
    mesh = plsc.VectorSubcoreMesh(core_axis_name="c", subcore_axis_name="s")
    f = functools.partial(
        pl.kernel,
        out_type=jax.ShapeDtypeStruct((_NW, 16), jnp.float32),
        mesh=mesh,
        scratch_types=[
            pltpu.VMEM((_FPW + 16,), jnp.float32),
            pltpu.VMEM((_FPW + 16,), jnp.float32),
            pltpu.VMEM((_RPW * _NP + 16,), jnp.float32),
            pltpu.VMEM((_RPW * _NP + 16,), jnp.float32),
            pltpu.VMEM((16,), jnp.float32),
        ],
    )(_sc_body)
    return f(similarities.reshape(-1), levs.reshape(-1))


def kernel(similarities, levs):
    levs = levs.reshape(similarities.shape)
    partials = _sc_pairwise(similarities, levs)
    return jnp.sum(partials) / jnp.float32(_B * _N * _N)


# speedup vs baseline: 2.4234x; 1.1149x over previous
import functools

import jax
import jax.numpy as jnp
from jax import lax
from jax.experimental import pallas as pl
from jax.experimental.pallas import tpu as pltpu
from jax.experimental.pallas import tpu_sc as plsc

SIGMA = 1.0

_B = 1024
_N = 200
_NC = 2
_NS = 16
_NW = _NC * _NS
_RPW = _B // _NW
_FPW = _RPW * _N
_NCHUNK = 13
_NP = _NCHUNK * 16
_INF = float("inf")


def _sc_body(sims_hbm, levs_hbm, out_hbm, ss_v, sl_v, d_v, l_v, o_v):
    wid = lax.axis_index("s") * _NC + lax.axis_index("c")
    base = wid * _FPW

    pltpu.sync_copy(sims_hbm.at[pl.ds(base, _FPW)], ss_v.at[pl.ds(0, _FPW)])
    pltpu.sync_copy(levs_hbm.at[pl.ds(base, _FPW)], sl_v.at[pl.ds(0, _FPW)])

    iota = lax.iota(jnp.int32, 16)
    head8 = iota < 8

    def lay_row(r, carry):
        src = r * _N
        dst = r * _NP
        for c in range(12):
            sv = ss_v[pl.ds(src + 16 * c, 16)]
            lv = sl_v[pl.ds(src + 16 * c, 16)]
            d_v[pl.ds(dst + 16 * c, 16)] = sv - SIGMA * lv
            l_v[pl.ds(dst + 16 * c, 16)] = lv
        sv = ss_v[pl.ds(src + 192, 16)]
        lv = sl_v[pl.ds(src + 192, 16)]
        d_v[pl.ds(dst + 192, 16)] = jnp.where(head8, sv - SIGMA * lv, _INF)
        l_v[pl.ds(dst + 192, 16)] = jnp.where(head8, lv, _INF)
        return carry
    lax.fori_loop(0, _RPW, lay_row, 0)

    zero16 = jnp.zeros((16,), jnp.float32)

    def row_body(r, accs):
        rbase = r * _NP
        dqs = [d_v[pl.ds(rbase + 16 * c, 16)] for c in range(_NCHUNK)]
        lqs = [l_v[pl.ds(rbase + 16 * c, 16)] for c in range(_NCHUNK)]

        for cp in range(_NCHUNK):
            def i_body(i, accs, cp=cp):
                pa = rbase + 16 * cp + i
                dp = jnp.full((16,), d_v[pl.ds(pa, 16)][0], jnp.float32)
                lp = jnp.full((16,), l_v[pl.ds(pa, 16)][0], jnp.float32)
                qmask = iota > jnp.full((16,), i, jnp.int32)
                new = list(accs)
                t = dp - dqs[cp]
                v = jnp.where(lp <= lqs[cp], t, -t)
                c = jnp.maximum(v, 0.0)
                new[cp] = new[cp] + jnp.where(qmask, c, zero16)
                for cq in range(cp + 1, _NCHUNK):
                    t = dp - dqs[cq]
                    v = jnp.where(lp <= lqs[cq], t, -t)
                    new[cq] = new[cq] + jnp.maximum(v, 0.0)
                return tuple(new)
            accs = lax.fori_loop(0, 16, i_body, accs)
        return accs

    accs = lax.fori_loop(0, _RPW, row_body, (zero16,) * _NCHUNK)
    total = accs[0]
    for c in range(1, _NCHUNK):
        total = total + accs[c]
    o_v[...] = total
    pltpu.sync_copy(o_v, out_hbm.at[wid])


@jax.jit
def _sc_pairwise(similarities, levs):
    mesh = plsc.VectorSubcoreMesh(core_axis_name="c", subcore_axis_name="s")
    f = functools.partial(
        pl.kernel,
        out_type=jax.ShapeDtypeStruct((_NW, 16), jnp.float32),
        mesh=mesh,
        scratch_types=[
            pltpu.VMEM((_FPW + 16,), jnp.float32),
            pltpu.VMEM((_FPW + 16,), jnp.float32),
            pltpu.VMEM((_RPW * _NP + 16,), jnp.float32),
            pltpu.VMEM((_RPW * _NP + 16,), jnp.float32),
            pltpu.VMEM((16,), jnp.float32),
        ],
    )(_sc_body)
    return f(similarities.reshape(-1), levs.reshape(-1))


def kernel(similarities, levs):
    levs = levs.reshape(similarities.shape)
    partials = _sc_pairwise(similarities, levs)
    return jnp.sum(partials) / jnp.float32(_B * _N * _N)
